# baseline (speedup 1.0000x reference)
import functools

import jax
import jax.numpy as jnp
from jax import lax
from jax.experimental import pallas as pl
from jax.experimental.pallas import tpu as pltpu
from jax.experimental.pallas import tpu_sc as plsc

POOL = 7
CC = 64
LANES = 16
NW = 32
NG = CC // LANES
GSTEP = 128
MAXPIX = 1024
NSTEPS = MAXPIX // GSTEP


def _roi_pool_sc(fm8, params, idx):
    nroi = params.shape[0]
    nchunk = idx.shape[1]
    rois_per_w = nroi // NW

    mesh = plsc.VectorSubcoreMesh(core_axis_name="c", subcore_axis_name="s")

    @functools.partial(
        pl.kernel,
        out_type=jax.ShapeDtypeStruct((nroi, POOL, POOL, nchunk * CC),
                                      jnp.float32),
        mesh=mesh,
        compiler_params=pltpu.CompilerParams(use_tc_tiling_on_sc=False),
        scratch_types=[
            pltpu.VMEM((MAXPIX, CC), jnp.float32),
            pltpu.VMEM((POOL, POOL, CC), jnp.float32),
            pltpu.VMEM((nroi, 2 * LANES), jnp.int32),
            pltpu.VMEM((nchunk, NSTEPS, GSTEP), jnp.int32),
            pltpu.SemaphoreType.DMA,
        ],
    )
    def k(fm8_hbm, params_hbm, idx_hbm, out_hbm,
          buf_v, out_v, par_v, idx_v, sem):
        wid = lax.axis_index("s") * 2 + lax.axis_index("c")
        pltpu.sync_copy(params_hbm, par_v)

        def roi_body(i, carry):
            roi = wid * rois_per_w + i
            vec0 = par_v[roi, pl.ds(0, LANES)]
            vec1 = par_v[roi, pl.ds(LANES, LANES)]
            ww = vec1[14]
            nsteps = vec1[15]
            pltpu.sync_copy(idx_hbm.at[roi], idx_v)

            def cc_body(ci, carry2):
                def gather_body(s, carry3):
                    pltpu.async_copy(
                        fm8_hbm.at[idx_v.at[ci, s]],
                        buf_v.at[pl.ds(s * GSTEP, GSTEP)],
                        sem).wait()
                    return carry3

                lax.fori_loop(0, nsteps, gather_body, 0)

                for py in range(POOL):
                    r0 = vec0[2 + py]
                    rn = vec0[9 + py]
                    for px in range(POOL):
                        c0 = vec1[px]
                        cn = vec1[POOL + px]

                        def row_body(r, accs):
                            base = r * ww + c0

                            def col_body(c, accs2):
                                return tuple(
                                    jnp.maximum(
                                        accs2[g],
                                        buf_v[c, pl.ds(g * LANES, LANES)])
                                    for g in range(NG))
                            return lax.fori_loop(base, base + cn, col_body,
                                                 accs)

                        neg = jnp.full((LANES,), -jnp.inf, jnp.float32)
                        accs = lax.fori_loop(r0, r0 + rn, row_body, (neg,) * NG)
                        for g in range(NG):
                            out_v[py, px, pl.ds(g * LANES, LANES)] = accs[g]

                pltpu.sync_copy(out_v,
                                out_hbm.at[roi, :, :, pl.ds(ci * CC, CC)])
                return carry2

            lax.fori_loop(0, nchunk, cc_body, 0)
            return carry

        lax.fori_loop(0, rois_per_w, roi_body, 0)

    return k(fm8, params, idx)


def kernel(x_maps, x_rois):
    B, H, W, C = x_maps.shape
    R = x_rois.shape[1]
    nchunk = C // CC
    y = x_rois[..., 0].astype(jnp.int32).reshape(-1)
    x = x_rois[..., 1].astype(jnp.int32).reshape(-1)
    h = x_rois[..., 2].astype(jnp.int32).reshape(-1)
    w = x_rois[..., 3].astype(jnp.int32).reshape(-1)
    nroi = B * R
    b = jnp.arange(nroi, dtype=jnp.int32) // R

    p = jnp.arange(POOL, dtype=jnp.int32)
    y0 = (p * h[:, None]) // POOL
    y1 = ((p + 1) * h[:, None]) // POOL
    ys = jnp.maximum(y1 - y0, 1)
    x0 = (p * w[:, None]) // POOL
    x1 = ((p + 1) * w[:, None]) // POOL
    xs = jnp.maximum(x1 - x0, 1)

    n = h * w
    nsteps = (n + GSTEP - 1) // GSTEP
    params = jnp.concatenate(
        [y[:, None], x[:, None], y0, ys, x0, xs,
         w[:, None], nsteps[:, None]],
        axis=-1).astype(jnp.int32)

    s = jnp.arange(MAXPIX, dtype=jnp.int32)
    pr = s[None, :] // w[:, None]
    pc = s[None, :] - pr * w[:, None]
    pix = ((b + 0)[:, None] * H + y[:, None] + pr) * W + x[:, None] + pc
    pix0 = ((b * H + y) * W + x)[:, None]
    pix = jnp.where(s[None, :] < n[:, None], pix, pix0)
    ci = jnp.arange(nchunk, dtype=jnp.int32)
    idx = (pix[:, None, :] * nchunk + ci[None, :, None]).astype(jnp.int32)
    idx = idx.reshape(nroi, nchunk, NSTEPS, GSTEP)

    fm8 = x_maps.reshape(B * H * W * nchunk, CC)
    out = _roi_pool_sc(fm8, params, idx)
    return out.reshape(B, R, POOL, POOL, C)


# speedup vs baseline: 3.9155x; 3.9155x over previous
import functools

import jax
import jax.numpy as jnp
from jax import lax
from jax.experimental import pallas as pl
from jax.experimental.pallas import tpu as pltpu
from jax.experimental.pallas import tpu_sc as plsc

POOL = 7
LANES = 16
NW = 32
MAXBAND = 5
CH = 256
NGH = CH // LANES
WCLS = (8, 16, 24, 32)


def _roi_pool_sc(fm3, ttab, nroi, c_total):
    ntask = ttab.shape[0]
    tpw = ntask // NW
    nunit = 2 * tpw

    mesh = plsc.VectorSubcoreMesh(core_axis_name="c", subcore_axis_name="s")

    @functools.partial(
        pl.kernel,
        out_type=jax.ShapeDtypeStruct((ntask, POOL, c_total), jnp.float32),
        mesh=mesh,
        compiler_params=pltpu.CompilerParams(use_tc_tiling_on_sc=False),
        scratch_types=[
            pltpu.VMEM((2, MAXBAND, 32, CH), jnp.float32),
            pltpu.VMEM((2, POOL, c_total), jnp.float32),
            pltpu.VMEM((tpw, 2 * LANES), jnp.int32),
            pltpu.SemaphoreType.DMA((2,)),
            pltpu.SemaphoreType.DMA,
        ],
    )
    def k(fm3_hbm, ttab_hbm, out_hbm, band_v, outr_v, tt_v, sem, osem):
        wid = lax.axis_index("s") * 2 + lax.axis_index("c")
        pltpu.sync_copy(ttab_hbm.at[pl.ds(wid * tpw, tpw)], tt_v)

        def fire(u):
            kb = u // 2
            half = u % 2
            par = u % 2
            vec = tt_v[kb, pl.ds(0, LANES)]
            rowstart = vec[0]
            rn = vec[1]
            wcls = vec[2]
            xcol = vec[4]
            for ic, wc in enumerate(WCLS):
                @pl.when(wcls == ic)
                def _():
                    def jb(j, c):
                        pltpu.async_copy(
                            fm3_hbm.at[rowstart + j, pl.ds(xcol, wc),
                                       pl.ds(half * CH, CH)],
                            band_v.at[par, j, pl.ds(0, wc)],
                            sem.at[par])
                        return c
                    lax.fori_loop(0, rn, jb, 0)

        def drain(u):
            kb = u // 2
            half = u % 2
            par = u % 2
            vec = tt_v[kb, pl.ds(0, LANES)]
            rowstart = vec[0]
            rn = vec[1]
            wcls = vec[2]
            xcol = vec[4]
            for ic, wc in enumerate(WCLS):
                @pl.when(wcls == ic)
                def _():
                    def jb(j, c):
                        pltpu.make_async_copy(
                            fm3_hbm.at[rowstart + j, pl.ds(xcol, wc),
                                       pl.ds(half * CH, CH)],
                            band_v.at[par, j, pl.ds(0, wc)],
                            sem.at[par]).wait()
                        return c
                    lax.fori_loop(0, rn, jb, 0)

        fire(0)

        def body(u, carry):
            kb = u // 2
            half = u % 2
            par = u % 2
            kpar = kb % 2

            @pl.when(u + 1 < nunit)
            def _():
                fire(u + 1)

            drain(u)

            vec0 = tt_v[kb, pl.ds(0, LANES)]
            vec1 = tt_v[kb, pl.ds(LANES, LANES)]
            rn = vec0[1]
            outpos = vec0[3]

            @pl.when((half == 0) & (kb >= 2))
            def _():
                prev = tt_v[jnp.maximum(kb - 2, 0), pl.ds(0, LANES)]
                pltpu.make_async_copy(
                    outr_v.at[kpar], out_hbm.at[prev[3]], osem).wait()

            for px in range(POOL):
                c0 = vec1[px]
                cn = vec1[POOL + px]

                def row_body(j, accs):
                    def col_body(c, accs2):
                        return tuple(
                            jnp.maximum(
                                accs2[g],
                                band_v[par, j, c, pl.ds(g * LANES, LANES)])
                            for g in range(NGH))
                    return lax.fori_loop(c0, c0 + cn, col_body, accs)

                neg = jnp.full((LANES,), -jnp.inf, jnp.float32)
                accs = lax.fori_loop(0, rn, row_body, (neg,) * NGH)
                for g in range(NGH):
                    outr_v[kpar, px, pl.ds(half * CH + g * LANES, LANES)] = \
                        accs[g]

            @pl.when(half == 1)
            def _():
                pltpu.async_copy(outr_v.at[kpar], out_hbm.at[outpos], osem)

            return carry

        lax.fori_loop(0, nunit, body, 0)

        def tail(t, carry):
            kb = tpw - 2 + t
            vec = tt_v[kb, pl.ds(0, LANES)]
            pltpu.make_async_copy(
                outr_v.at[kb % 2], out_hbm.at[vec[3]], osem).wait()
            return carry

        lax.fori_loop(0, 2, tail, 0)

    return k(fm3, ttab)


def kernel(x_maps, x_rois):
    B, H, W, C = x_maps.shape
    R = x_rois.shape[1]
    nroi = B * R
    y = x_rois[..., 0].astype(jnp.int32).reshape(-1)
    x = x_rois[..., 1].astype(jnp.int32).reshape(-1)
    h = x_rois[..., 2].astype(jnp.int32).reshape(-1)
    w = x_rois[..., 3].astype(jnp.int32).reshape(-1)
    b = jnp.arange(nroi, dtype=jnp.int32) // R

    p = jnp.arange(POOL, dtype=jnp.int32)
    y0 = (p * h[:, None]) // POOL
    y1 = ((p + 1) * h[:, None]) // POOL
    ys = jnp.maximum(y1 - y0, 1)
    x0 = (p * w[:, None]) // POOL
    x1 = ((p + 1) * w[:, None]) // POOL
    xs = jnp.maximum(x1 - x0, 1)

    ntask = nroi * POOL
    rowstart = ((b * H + y)[:, None] + y0)
    rn = ys
    wcls = jnp.minimum((w + 7) // 8, 4) - 1
    outpos = (jnp.arange(nroi, dtype=jnp.int32)[:, None] * POOL + p)
    zero = jnp.zeros((nroi, POOL), jnp.int32)

    def bcast(a):
        return jnp.broadcast_to(a[:, None], (nroi, POOL))

    vec0 = jnp.stack(
        [rowstart, rn, bcast(wcls), outpos, bcast(x)]
        + [zero] * 11, axis=-1)
    vec1 = jnp.concatenate(
        [jnp.broadcast_to(x0[:, None, :], (nroi, POOL, POOL)),
         jnp.broadcast_to(xs[:, None, :], (nroi, POOL, POOL)),
         jnp.zeros((nroi, POOL, 2), jnp.int32)], axis=-1)
    ttab = jnp.concatenate([vec0, vec1], axis=-1).reshape(ntask, 2 * LANES)

    cost = (rn * ((w[:, None] + 7) // 8 * 8)).reshape(ntask)
    ranks = jnp.argsort(-cost)
    perm = ranks.reshape(ntask // NW, NW).T.reshape(ntask)
    ttab = ttab[perm].astype(jnp.int32)

    fm3 = x_maps.reshape(B * H, W, C)
    out = _roi_pool_sc(fm3, ttab, nroi, C)

    return out.reshape(B, R, POOL, POOL, C)
